# baseline (speedup 1.0000x reference)
import jax
import jax.numpy as jnp
from jax.experimental import pallas as pl
from jax.experimental.pallas import tpu as pltpu

N = 8192
INPUT_D = 256
HIDDEN_D = 128
OUTPUT_D = 64
ALP = 0.5
LAM = 1.0

BM = 512
BK = 512


def _h_kernel(x_ref, w_ref, b_ref, h_ref):
    h_ref[...] = (
        jnp.dot(x_ref[...], w_ref[...], preferred_element_type=jnp.float32)
        + b_ref[...]
    )


def _pass1_kernel(a_ref, d_ref, hk_ref, hi_ref, y1_ref, dh_ref, acc_a, acc_d):
    k = pl.program_id(1)

    @pl.when(k == 0)
    def _():
        acc_a[...] = jnp.zeros_like(acc_a)
        acc_d[...] = jnp.zeros_like(acc_d)

    hk = hk_ref[...]
    acc_a[...] += jnp.dot(a_ref[...], hk, preferred_element_type=jnp.float32)
    acc_d[...] += jnp.dot(d_ref[...], hk, preferred_element_type=jnp.float32)

    @pl.when(k == pl.num_programs(1) - 1)
    def _():
        dh = acc_d[...]
        dh_ref[...] = dh
        y1_ref[...] = (1.0 - ALP) * hi_ref[...] + (ALP * LAM) * acc_a[...] + ALP * dh


def _pass2_kernel(a_ref, yk_ref, yi_ref, dh_ref, w_ref, b_ref, out_ref, acc):
    k = pl.program_id(1)

    @pl.when(k == 0)
    def _():
        acc[...] = jnp.zeros_like(acc)

    acc[...] += jnp.dot(a_ref[...], yk_ref[...], preferred_element_type=jnp.float32)

    @pl.when(k == pl.num_programs(1) - 1)
    def _():
        y2 = (
            (1.0 - ALP) * yi_ref[...]
            + (ALP * LAM) * acc[...]
            + ALP * dh_ref[...]
        )
        z = jnp.maximum(y2, 0.0)
        out_ref[...] = (
            jnp.dot(z, w_ref[...], preferred_element_type=jnp.float32) + b_ref[...]
        )


def kernel(x, sem_adj, norm_diag, W_bef, b_bef, W_aft, b_aft):
    h = pl.pallas_call(
        _h_kernel,
        out_shape=jax.ShapeDtypeStruct((N, HIDDEN_D), jnp.float32),
    )(x, W_bef, b_bef.reshape(1, HIDDEN_D))

    grid = (N // BM, N // BK)
    y1, dh = pl.pallas_call(
        _pass1_kernel,
        grid=grid,
        in_specs=[
            pl.BlockSpec((BM, BK), lambda i, k: (i, k)),
            pl.BlockSpec((BM, BK), lambda i, k: (i, k)),
            pl.BlockSpec((BK, HIDDEN_D), lambda i, k: (k, 0)),
            pl.BlockSpec((BM, HIDDEN_D), lambda i, k: (i, 0)),
        ],
        out_specs=[
            pl.BlockSpec((BM, HIDDEN_D), lambda i, k: (i, 0)),
            pl.BlockSpec((BM, HIDDEN_D), lambda i, k: (i, 0)),
        ],
        out_shape=[
            jax.ShapeDtypeStruct((N, HIDDEN_D), jnp.float32),
            jax.ShapeDtypeStruct((N, HIDDEN_D), jnp.float32),
        ],
        scratch_shapes=[
            pltpu.VMEM((BM, HIDDEN_D), jnp.float32),
            pltpu.VMEM((BM, HIDDEN_D), jnp.float32),
        ],
        compiler_params=pltpu.CompilerParams(
            dimension_semantics=("parallel", "arbitrary"),
        ),
    )(sem_adj, norm_diag, h, h)

    out = pl.pallas_call(
        _pass2_kernel,
        grid=grid,
        in_specs=[
            pl.BlockSpec((BM, BK), lambda i, k: (i, k)),
            pl.BlockSpec((BK, HIDDEN_D), lambda i, k: (k, 0)),
            pl.BlockSpec((BM, HIDDEN_D), lambda i, k: (i, 0)),
            pl.BlockSpec((BM, HIDDEN_D), lambda i, k: (i, 0)),
            pl.BlockSpec((HIDDEN_D, OUTPUT_D), lambda i, k: (0, 0)),
            pl.BlockSpec((1, OUTPUT_D), lambda i, k: (0, 0)),
        ],
        out_specs=pl.BlockSpec((BM, OUTPUT_D), lambda i, k: (i, 0)),
        out_shape=jax.ShapeDtypeStruct((N, OUTPUT_D), jnp.float32),
        scratch_shapes=[pltpu.VMEM((BM, HIDDEN_D), jnp.float32)],
        compiler_params=pltpu.CompilerParams(
            dimension_semantics=("parallel", "arbitrary"),
        ),
    )(sem_adj, y1, y1, dh, W_aft, b_aft.reshape(1, OUTPUT_D))

    return out


# speedup vs baseline: 1.0027x; 1.0027x over previous
import jax
import jax.numpy as jnp
from jax.experimental import pallas as pl
from jax.experimental.pallas import tpu as pltpu

N = 8192
INPUT_D = 256
HIDDEN_D = 128
OUTPUT_D = 64
ALP = 0.5
LAM = 1.0

BM = 512
BK = 512


def _h_kernel(x_ref, w_ref, b_ref, h_ref):
    h_ref[...] = (
        jnp.dot(x_ref[...], w_ref[...], preferred_element_type=jnp.float32)
        + b_ref[...]
    )


def _pass1_kernel(a_ref, d_ref, hk_ref, hi_ref, y1_ref, dh_ref, acc_a, acc_d):
    k = pl.program_id(1)

    @pl.when(k == 0)
    def _():
        acc_a[...] = jnp.zeros_like(acc_a)
        acc_d[...] = jnp.zeros_like(acc_d)

    hk = hk_ref[...].astype(jnp.bfloat16)
    a = a_ref[...].astype(jnp.bfloat16)
    d = d_ref[...].astype(jnp.bfloat16)
    acc_a[...] += jnp.dot(a, hk, preferred_element_type=jnp.float32)
    acc_d[...] += jnp.dot(d, hk, preferred_element_type=jnp.float32)

    @pl.when(k == pl.num_programs(1) - 1)
    def _():
        dh = acc_d[...]
        dh_ref[...] = dh
        y1_ref[...] = (1.0 - ALP) * hi_ref[...] + (ALP * LAM) * acc_a[...] + ALP * dh


def _pass2_kernel(a_ref, yk_ref, yi_ref, dh_ref, w_ref, b_ref, out_ref, acc):
    k = pl.program_id(1)

    @pl.when(k == 0)
    def _():
        acc[...] = jnp.zeros_like(acc)

    acc[...] += jnp.dot(
        a_ref[...].astype(jnp.bfloat16),
        yk_ref[...].astype(jnp.bfloat16),
        preferred_element_type=jnp.float32,
    )

    @pl.when(k == pl.num_programs(1) - 1)
    def _():
        y2 = (
            (1.0 - ALP) * yi_ref[...]
            + (ALP * LAM) * acc[...]
            + ALP * dh_ref[...]
        )
        z = jnp.maximum(y2, 0.0)
        out_ref[...] = (
            jnp.dot(z, w_ref[...], preferred_element_type=jnp.float32) + b_ref[...]
        )


def kernel(x, sem_adj, norm_diag, W_bef, b_bef, W_aft, b_aft):
    h = pl.pallas_call(
        _h_kernel,
        out_shape=jax.ShapeDtypeStruct((N, HIDDEN_D), jnp.float32),
    )(x, W_bef, b_bef.reshape(1, HIDDEN_D))

    grid = (N // BM, N // BK)
    y1, dh = pl.pallas_call(
        _pass1_kernel,
        grid=grid,
        in_specs=[
            pl.BlockSpec((BM, BK), lambda i, k: (i, k)),
            pl.BlockSpec((BM, BK), lambda i, k: (i, k)),
            pl.BlockSpec((BK, HIDDEN_D), lambda i, k: (k, 0)),
            pl.BlockSpec((BM, HIDDEN_D), lambda i, k: (i, 0)),
        ],
        out_specs=[
            pl.BlockSpec((BM, HIDDEN_D), lambda i, k: (i, 0)),
            pl.BlockSpec((BM, HIDDEN_D), lambda i, k: (i, 0)),
        ],
        out_shape=[
            jax.ShapeDtypeStruct((N, HIDDEN_D), jnp.float32),
            jax.ShapeDtypeStruct((N, HIDDEN_D), jnp.float32),
        ],
        scratch_shapes=[
            pltpu.VMEM((BM, HIDDEN_D), jnp.float32),
            pltpu.VMEM((BM, HIDDEN_D), jnp.float32),
        ],
        compiler_params=pltpu.CompilerParams(
            dimension_semantics=("parallel", "arbitrary"),
        ),
    )(sem_adj, norm_diag, h, h)

    out = pl.pallas_call(
        _pass2_kernel,
        grid=grid,
        in_specs=[
            pl.BlockSpec((BM, BK), lambda i, k: (i, k)),
            pl.BlockSpec((BK, HIDDEN_D), lambda i, k: (k, 0)),
            pl.BlockSpec((BM, HIDDEN_D), lambda i, k: (i, 0)),
            pl.BlockSpec((BM, HIDDEN_D), lambda i, k: (i, 0)),
            pl.BlockSpec((HIDDEN_D, OUTPUT_D), lambda i, k: (0, 0)),
            pl.BlockSpec((1, OUTPUT_D), lambda i, k: (0, 0)),
        ],
        out_specs=pl.BlockSpec((BM, OUTPUT_D), lambda i, k: (i, 0)),
        out_shape=jax.ShapeDtypeStruct((N, OUTPUT_D), jnp.float32),
        scratch_shapes=[pltpu.VMEM((BM, HIDDEN_D), jnp.float32)],
        compiler_params=pltpu.CompilerParams(
            dimension_semantics=("parallel", "arbitrary"),
        ),
    )(sem_adj, y1, y1, dh, W_aft, b_aft.reshape(1, OUTPUT_D))

    return out


# speedup vs baseline: 1.7389x; 1.7343x over previous
import jax
import jax.numpy as jnp
from jax.experimental import pallas as pl
from jax.experimental.pallas import tpu as pltpu

N = 8192
INPUT_D = 256
HIDDEN_D = 128
OUTPUT_D = 64
ALP = 0.5
LAM = 1.0

BM = 512
BK = 2048


def _h_kernel(x_ref, w_ref, b_ref, h_ref):
    h_ref[...] = (
        jnp.dot(x_ref[...], w_ref[...], preferred_element_type=jnp.float32)
        + b_ref[...]
    )


def _pass1_kernel(a_ref, d_ref, hk_ref, hi_ref, y1_ref, dh_ref, acc_a, acc_d):
    k = pl.program_id(1)

    @pl.when(k == 0)
    def _():
        acc_a[...] = jnp.zeros_like(acc_a)
        acc_d[...] = jnp.zeros_like(acc_d)

    hk = hk_ref[...].astype(jnp.bfloat16)
    a = a_ref[...].astype(jnp.bfloat16)
    d = d_ref[...].astype(jnp.bfloat16)
    acc_a[...] += jnp.dot(a, hk, preferred_element_type=jnp.float32)
    acc_d[...] += jnp.dot(d, hk, preferred_element_type=jnp.float32)

    @pl.when(k == pl.num_programs(1) - 1)
    def _():
        dh = acc_d[...]
        dh_ref[...] = dh
        y1_ref[...] = (1.0 - ALP) * hi_ref[...] + (ALP * LAM) * acc_a[...] + ALP * dh


def _pass2_kernel(a_ref, yk_ref, yi_ref, dh_ref, w_ref, b_ref, out_ref, acc):
    k = pl.program_id(1)

    @pl.when(k == 0)
    def _():
        acc[...] = jnp.zeros_like(acc)

    acc[...] += jnp.dot(
        a_ref[...].astype(jnp.bfloat16),
        yk_ref[...].astype(jnp.bfloat16),
        preferred_element_type=jnp.float32,
    )

    @pl.when(k == pl.num_programs(1) - 1)
    def _():
        y2 = (
            (1.0 - ALP) * yi_ref[...]
            + (ALP * LAM) * acc[...]
            + ALP * dh_ref[...]
        )
        z = jnp.maximum(y2, 0.0)
        out_ref[...] = (
            jnp.dot(z, w_ref[...], preferred_element_type=jnp.float32) + b_ref[...]
        )


def kernel(x, sem_adj, norm_diag, W_bef, b_bef, W_aft, b_aft):
    h = pl.pallas_call(
        _h_kernel,
        out_shape=jax.ShapeDtypeStruct((N, HIDDEN_D), jnp.float32),
    )(x, W_bef, b_bef.reshape(1, HIDDEN_D))

    grid = (N // BM, N // BK)
    y1, dh = pl.pallas_call(
        _pass1_kernel,
        grid=grid,
        in_specs=[
            pl.BlockSpec((BM, BK), lambda i, k: (i, k)),
            pl.BlockSpec((BM, BK), lambda i, k: (i, k)),
            pl.BlockSpec((BK, HIDDEN_D), lambda i, k: (k, 0)),
            pl.BlockSpec((BM, HIDDEN_D), lambda i, k: (i, 0)),
        ],
        out_specs=[
            pl.BlockSpec((BM, HIDDEN_D), lambda i, k: (i, 0)),
            pl.BlockSpec((BM, HIDDEN_D), lambda i, k: (i, 0)),
        ],
        out_shape=[
            jax.ShapeDtypeStruct((N, HIDDEN_D), jnp.float32),
            jax.ShapeDtypeStruct((N, HIDDEN_D), jnp.float32),
        ],
        scratch_shapes=[
            pltpu.VMEM((BM, HIDDEN_D), jnp.float32),
            pltpu.VMEM((BM, HIDDEN_D), jnp.float32),
        ],
        compiler_params=pltpu.CompilerParams(
            dimension_semantics=("parallel", "arbitrary"),
        ),
    )(sem_adj, norm_diag, h, h)

    out = pl.pallas_call(
        _pass2_kernel,
        grid=grid,
        in_specs=[
            pl.BlockSpec((BM, BK), lambda i, k: (i, k)),
            pl.BlockSpec((BK, HIDDEN_D), lambda i, k: (k, 0)),
            pl.BlockSpec((BM, HIDDEN_D), lambda i, k: (i, 0)),
            pl.BlockSpec((BM, HIDDEN_D), lambda i, k: (i, 0)),
            pl.BlockSpec((HIDDEN_D, OUTPUT_D), lambda i, k: (0, 0)),
            pl.BlockSpec((1, OUTPUT_D), lambda i, k: (0, 0)),
        ],
        out_specs=pl.BlockSpec((BM, OUTPUT_D), lambda i, k: (i, 0)),
        out_shape=jax.ShapeDtypeStruct((N, OUTPUT_D), jnp.float32),
        scratch_shapes=[pltpu.VMEM((BM, HIDDEN_D), jnp.float32)],
        compiler_params=pltpu.CompilerParams(
            dimension_semantics=("parallel", "arbitrary"),
        ),
    )(sem_adj, y1, y1, dh, W_aft, b_aft.reshape(1, OUTPUT_D))

    return out


# speedup vs baseline: 1.7832x; 1.0254x over previous
import jax
import jax.numpy as jnp
from jax.experimental import pallas as pl
from jax.experimental.pallas import tpu as pltpu

N = 8192
INPUT_D = 256
HIDDEN_D = 128
OUTPUT_D = 64
ALP = 0.5
LAM = 1.0

BM = 512
BK = 4096


def _h_kernel(x_ref, w_ref, b_ref, h_ref):
    h_ref[...] = (
        jnp.dot(x_ref[...], w_ref[...], preferred_element_type=jnp.float32)
        + b_ref[...]
    )


def _pass1_kernel(a_ref, d_ref, hk_ref, hi_ref, y1_ref, dh_ref, acc_a, acc_d):
    k = pl.program_id(1)

    @pl.when(k == 0)
    def _():
        acc_a[...] = jnp.zeros_like(acc_a)
        acc_d[...] = jnp.zeros_like(acc_d)

    hk = hk_ref[...].astype(jnp.bfloat16)
    a = a_ref[...].astype(jnp.bfloat16)
    d = d_ref[...].astype(jnp.bfloat16)
    acc_a[...] += jnp.dot(a, hk, preferred_element_type=jnp.float32)
    acc_d[...] += jnp.dot(d, hk, preferred_element_type=jnp.float32)

    @pl.when(k == pl.num_programs(1) - 1)
    def _():
        dh = acc_d[...]
        dh_ref[...] = dh
        y1_ref[...] = (1.0 - ALP) * hi_ref[...] + (ALP * LAM) * acc_a[...] + ALP * dh


def _pass2_kernel(a_ref, yk_ref, yi_ref, dh_ref, w_ref, b_ref, out_ref, acc):
    k = pl.program_id(1)

    @pl.when(k == 0)
    def _():
        acc[...] = jnp.zeros_like(acc)

    acc[...] += jnp.dot(
        a_ref[...].astype(jnp.bfloat16),
        yk_ref[...].astype(jnp.bfloat16),
        preferred_element_type=jnp.float32,
    )

    @pl.when(k == pl.num_programs(1) - 1)
    def _():
        y2 = (
            (1.0 - ALP) * yi_ref[...]
            + (ALP * LAM) * acc[...]
            + ALP * dh_ref[...]
        )
        z = jnp.maximum(y2, 0.0)
        out_ref[...] = (
            jnp.dot(z, w_ref[...], preferred_element_type=jnp.float32) + b_ref[...]
        )


def kernel(x, sem_adj, norm_diag, W_bef, b_bef, W_aft, b_aft):
    h = pl.pallas_call(
        _h_kernel,
        out_shape=jax.ShapeDtypeStruct((N, HIDDEN_D), jnp.float32),
    )(x, W_bef, b_bef.reshape(1, HIDDEN_D))

    grid = (N // BM, N // BK)
    y1, dh = pl.pallas_call(
        _pass1_kernel,
        grid=grid,
        in_specs=[
            pl.BlockSpec((BM, BK), lambda i, k: (i, k)),
            pl.BlockSpec((BM, BK), lambda i, k: (i, k)),
            pl.BlockSpec((BK, HIDDEN_D), lambda i, k: (k, 0)),
            pl.BlockSpec((BM, HIDDEN_D), lambda i, k: (i, 0)),
        ],
        out_specs=[
            pl.BlockSpec((BM, HIDDEN_D), lambda i, k: (i, 0)),
            pl.BlockSpec((BM, HIDDEN_D), lambda i, k: (i, 0)),
        ],
        out_shape=[
            jax.ShapeDtypeStruct((N, HIDDEN_D), jnp.float32),
            jax.ShapeDtypeStruct((N, HIDDEN_D), jnp.float32),
        ],
        scratch_shapes=[
            pltpu.VMEM((BM, HIDDEN_D), jnp.float32),
            pltpu.VMEM((BM, HIDDEN_D), jnp.float32),
        ],
        compiler_params=pltpu.CompilerParams(
            dimension_semantics=("parallel", "arbitrary"),
        ),
    )(sem_adj, norm_diag, h, h)

    out = pl.pallas_call(
        _pass2_kernel,
        grid=grid,
        in_specs=[
            pl.BlockSpec((BM, BK), lambda i, k: (i, k)),
            pl.BlockSpec((BK, HIDDEN_D), lambda i, k: (k, 0)),
            pl.BlockSpec((BM, HIDDEN_D), lambda i, k: (i, 0)),
            pl.BlockSpec((BM, HIDDEN_D), lambda i, k: (i, 0)),
            pl.BlockSpec((HIDDEN_D, OUTPUT_D), lambda i, k: (0, 0)),
            pl.BlockSpec((1, OUTPUT_D), lambda i, k: (0, 0)),
        ],
        out_specs=pl.BlockSpec((BM, OUTPUT_D), lambda i, k: (i, 0)),
        out_shape=jax.ShapeDtypeStruct((N, OUTPUT_D), jnp.float32),
        scratch_shapes=[pltpu.VMEM((BM, HIDDEN_D), jnp.float32)],
        compiler_params=pltpu.CompilerParams(
            dimension_semantics=("parallel", "arbitrary"),
        ),
    )(sem_adj, y1, y1, dh, W_aft, b_aft.reshape(1, OUTPUT_D))

    return out
